# baseline (speedup 1.0000x reference)
import functools

import jax
import jax.numpy as jnp
from jax import lax
from jax.experimental import pallas as pl
from jax.experimental.pallas import tpu as pltpu
from jax.experimental.pallas import tpu_sc as plsc

D = 768
H = D // 2
EPS = 1e-12
_NC, _NS = 2, 16
_NW = _NC * _NS
_CH = 32


def _sc_gather_pack(word_table, idx3):
    nw, n_ch, ch = idx3.shape
    b_total = nw * n_ch * ch
    mesh = plsc.VectorSubcoreMesh(core_axis_name="c", subcore_axis_name="s")

    @functools.partial(
        pl.kernel,
        mesh=mesh,
        out_type=jax.ShapeDtypeStruct((b_total, H), jnp.int32),
        scratch_types=[
            pltpu.VMEM((n_ch, ch), jnp.int32),
            pltpu.VMEM((ch, D), jnp.float32),
            pltpu.VMEM((ch, D), jnp.float32),
            pltpu.VMEM((ch, H), jnp.int32),
            pltpu.VMEM((ch, H), jnp.int32),
            pltpu.SemaphoreType.DMA,
            pltpu.SemaphoreType.DMA,
            pltpu.SemaphoreType.DMA,
            pltpu.SemaphoreType.DMA,
        ],
    )
    def k(table_hbm, idx_hbm, out_hbm, idx_v, gb0, gb1, ob0, ob1,
          gs0, gs1, os0, os1):
        wid = lax.axis_index("s") * _NC + lax.axis_index("c")
        base = wid * (n_ch * ch)
        pltpu.sync_copy(idx_hbm.at[wid], idx_v)
        gbufs = (gb0, gb1)
        obufs = (ob0, ob1)
        gsems = (gs0, gs1)
        osems = (os0, os1)

        pltpu.async_copy(table_hbm.at[idx_v.at[0]], gb0, gs0)
        if n_ch > 1:
            pltpu.async_copy(table_hbm.at[idx_v.at[1]], gb1, gs1)

        def chunk(i, b, c):
            gb, ob = gbufs[b], obufs[b]
            pltpu.make_async_copy(
                table_hbm.at[idx_v.at[c]], gb, gsems[b]).wait()

            @pl.when(i >= 1)
            def _():
                pltpu.make_async_copy(
                    ob, out_hbm.at[pl.ds(base + (c - 2) * ch, ch)],
                    osems[b]).wait()

            def row(r, carry):
                for m in range(H // 16):
                    a = gb[r, pl.ds(16 * m, 16)]
                    bv = gb[r, pl.ds(H + 16 * m, 16)]
                    ai = lax.bitcast_convert_type(a, jnp.int32) + 0x8000
                    bi = lax.bitcast_convert_type(bv, jnp.int32) + 0x8000
                    ob[r, pl.ds(16 * m, 16)] = (
                        lax.shift_right_logical(ai, 16)
                        | (bi & jnp.int32(-65536)))
                return carry

            lax.fori_loop(0, ch, row, 0)

            pltpu.async_copy(ob, out_hbm.at[pl.ds(base + c * ch, ch)],
                             osems[b])

            @pl.when(i < n_ch // 2 - 1)
            def _():
                pltpu.async_copy(table_hbm.at[idx_v.at[c + 2]], gb, gsems[b])

        def body(i, carry):
            chunk(i, 0, 2 * i)
            chunk(i, 1, 2 * i + 1)
            return carry

        lax.fori_loop(0, n_ch // 2, body, 0)

        pltpu.make_async_copy(
            ob0, out_hbm.at[pl.ds(base + (n_ch - 2) * ch, ch)], os0).wait()
        pltpu.make_async_copy(
            ob1, out_hbm.at[pl.ds(base + (n_ch - 1) * ch, ch)], os1).wait()

    return k(word_table, idx3)


def _ln_body(tt_ref, tab_ref, gamma_ref, w_ref, o_ref):
    w = w_ref[...]
    xa = lax.bitcast_convert_type(w << 16, jnp.float32)
    xb = lax.bitcast_convert_type(
        w & jnp.int32(-65536), jnp.float32)
    ttf = tt_ref[0]
    t0a = tab_ref[0, :H][None, :]
    t0b = tab_ref[0, H:][None, :]
    dta = tab_ref[1, :H][None, :] - t0a
    dtb = tab_ref[1, H:][None, :] - t0b
    xa = xa + t0a + ttf * dta
    xb = xb + t0b + ttf * dtb
    mean = (jnp.sum(xa, axis=1, keepdims=True) +
            jnp.sum(xb, axis=1, keepdims=True)) * (1.0 / D)
    msq = (jnp.sum(xa * xa, axis=1, keepdims=True) +
           jnp.sum(xb * xb, axis=1, keepdims=True)) * (1.0 / D)
    rs = lax.rsqrt(msq - mean * mean + EPS)
    bia = mean * rs
    o_ref[:, :H] = (xa * rs - bia) * gamma_ref[0, :H][None, :]
    o_ref[:, H:] = (xb * rs - bia) * gamma_ref[0, H:][None, :]


def _tc_layernorm(packed, token_type_flat, type_table, gamma, tb=2048):
    b_total = packed.shape[0]
    nb = b_total // tb
    tt3 = token_type_flat.reshape(nb, tb, 1).astype(jnp.float32)
    gamma2 = gamma.reshape(1, D)
    return pl.pallas_call(
        _ln_body,
        grid=(nb,),
        in_specs=[
            pl.BlockSpec((1, tb, 1), lambda i: (i, 0, 0)),
            pl.BlockSpec((2, D), lambda i: (0, 0)),
            pl.BlockSpec((1, D), lambda i: (0, 0)),
            pl.BlockSpec((tb, H), lambda i: (i, 0)),
        ],
        out_specs=pl.BlockSpec((tb, D), lambda i: (i, 0)),
        out_shape=jax.ShapeDtypeStruct((b_total, D), jnp.float32),
    )(tt3, type_table, gamma2, packed)


def kernel(input_ids, token_type_ids, word_table, type_table, gamma):
    batch, seq = input_ids.shape
    b_total = batch * seq
    n_ch = b_total // (_NW * _CH)
    idx3 = input_ids.reshape(_NW, n_ch, _CH)
    packed = _sc_gather_pack(word_table, idx3)
    out = _tc_layernorm(packed, token_type_ids.reshape(-1), type_table, gamma)
    return out.reshape(batch, seq, D)


# speedup vs baseline: 1.0599x; 1.0599x over previous
import functools

import jax
import jax.numpy as jnp
from jax import lax
from jax.experimental import pallas as pl
from jax.experimental.pallas import tpu as pltpu
from jax.experimental.pallas import tpu_sc as plsc

D = 768
H = D // 2
EPS = 1e-12
_NC, _NS = 2, 16
_NW = _NC * _NS
_CH = 32


def _sc_gather_pack(word_table, idx3):
    nw, n_ch, ch = idx3.shape
    b_total = nw * n_ch * ch
    mesh = plsc.VectorSubcoreMesh(core_axis_name="c", subcore_axis_name="s")

    @functools.partial(
        pl.kernel,
        mesh=mesh,
        out_type=jax.ShapeDtypeStruct((b_total, H), jnp.int32),
        scratch_types=[
            pltpu.VMEM((n_ch, ch), jnp.int32),
            pltpu.VMEM((ch, D), jnp.float32),
            pltpu.VMEM((ch, D), jnp.float32),
            pltpu.VMEM((ch, H), jnp.int32),
            pltpu.VMEM((ch, H), jnp.int32),
            pltpu.SemaphoreType.DMA,
            pltpu.SemaphoreType.DMA,
            pltpu.SemaphoreType.DMA,
            pltpu.SemaphoreType.DMA,
        ],
    )
    def k(table_hbm, idx_hbm, out_hbm, idx_v, gb0, gb1, ob0, ob1,
          gs0, gs1, os0, os1):
        wid = lax.axis_index("s") * _NC + lax.axis_index("c")
        base = wid * (n_ch * ch)
        pltpu.sync_copy(idx_hbm.at[wid], idx_v)
        gbufs = (gb0, gb1)
        obufs = (ob0, ob1)
        gsems = (gs0, gs1)
        osems = (os0, os1)

        pltpu.async_copy(table_hbm.at[idx_v.at[0]], gb0, gs0)
        if n_ch > 1:
            pltpu.async_copy(table_hbm.at[idx_v.at[1]], gb1, gs1)

        def chunk(i, b, c):
            gb, ob = gbufs[b], obufs[b]
            pltpu.make_async_copy(
                table_hbm.at[idx_v.at[c]], gb, gsems[b]).wait()

            @pl.when(i >= 1)
            def _():
                pltpu.make_async_copy(
                    ob, out_hbm.at[pl.ds(base + (c - 2) * ch, ch)],
                    osems[b]).wait()

            def row(i2, carry):
                for half in range(2):
                    r = 2 * i2 + half
                    for m in range(H // 16):
                        a = gb[r, pl.ds(16 * m, 16)]
                        bv = gb[r, pl.ds(H + 16 * m, 16)]
                        ai = lax.bitcast_convert_type(a, jnp.int32)
                        bi = lax.bitcast_convert_type(bv, jnp.int32)
                        ob[r, pl.ds(16 * m, 16)] = (
                            lax.shift_right_logical(ai, 16)
                            | (bi & jnp.int32(-65536)))
                return carry

            lax.fori_loop(0, ch // 2, row, 0)

            pltpu.async_copy(ob, out_hbm.at[pl.ds(base + c * ch, ch)],
                             osems[b])

            @pl.when(i < n_ch // 2 - 1)
            def _():
                pltpu.async_copy(table_hbm.at[idx_v.at[c + 2]], gb, gsems[b])

        def body(i, carry):
            chunk(i, 0, 2 * i)
            chunk(i, 1, 2 * i + 1)
            return carry

        lax.fori_loop(0, n_ch // 2, body, 0)

        pltpu.make_async_copy(
            ob0, out_hbm.at[pl.ds(base + (n_ch - 2) * ch, ch)], os0).wait()
        pltpu.make_async_copy(
            ob1, out_hbm.at[pl.ds(base + (n_ch - 1) * ch, ch)], os1).wait()

    return k(word_table, idx3)


def _ln_body(tt_ref, tab_ref, gamma_ref, w_ref, o_ref):
    w = w_ref[...]
    xa = lax.bitcast_convert_type(w << 16, jnp.float32)
    xb = lax.bitcast_convert_type(
        w & jnp.int32(-65536), jnp.float32)
    ttf = tt_ref[0]
    t0a = tab_ref[0, :H][None, :]
    t0b = tab_ref[0, H:][None, :]
    dta = tab_ref[1, :H][None, :] - t0a
    dtb = tab_ref[1, H:][None, :] - t0b
    xa = xa + t0a + ttf * dta
    xb = xb + t0b + ttf * dtb
    mean = (jnp.sum(xa, axis=1, keepdims=True) +
            jnp.sum(xb, axis=1, keepdims=True)) * (1.0 / D)
    msq = (jnp.sum(xa * xa, axis=1, keepdims=True) +
           jnp.sum(xb * xb, axis=1, keepdims=True)) * (1.0 / D)
    rs = lax.rsqrt(msq - mean * mean + EPS)
    bia = mean * rs
    o_ref[:, :H] = (xa * rs - bia) * gamma_ref[0, :H][None, :]
    o_ref[:, H:] = (xb * rs - bia) * gamma_ref[0, H:][None, :]


def _tc_layernorm(packed, token_type_flat, type_table, gamma, tb=2048):
    b_total = packed.shape[0]
    nb = b_total // tb
    tt3 = token_type_flat.reshape(nb, tb, 1).astype(jnp.float32)
    gamma2 = gamma.reshape(1, D)
    return pl.pallas_call(
        _ln_body,
        grid=(nb,),
        in_specs=[
            pl.BlockSpec((1, tb, 1), lambda i: (i, 0, 0)),
            pl.BlockSpec((2, D), lambda i: (0, 0)),
            pl.BlockSpec((1, D), lambda i: (0, 0)),
            pl.BlockSpec((tb, H), lambda i: (i, 0)),
        ],
        out_specs=pl.BlockSpec((tb, D), lambda i: (i, 0)),
        out_shape=jax.ShapeDtypeStruct((b_total, D), jnp.float32),
    )(tt3, type_table, gamma2, packed)


def kernel(input_ids, token_type_ids, word_table, type_table, gamma):
    batch, seq = input_ids.shape
    b_total = batch * seq
    n_ch = b_total // (_NW * _CH)
    idx3 = input_ids.reshape(_NW, n_ch, _CH)
    packed = _sc_gather_pack(word_table, idx3)
    out = _tc_layernorm(packed, token_type_ids.reshape(-1), type_table, gamma)
    return out.reshape(batch, seq, D)


# speedup vs baseline: 1.5653x; 1.4769x over previous
import functools

import jax
import jax.numpy as jnp
from jax import lax
from jax.experimental import pallas as pl
from jax.experimental.pallas import tpu as pltpu
from jax.experimental.pallas import tpu_sc as plsc

D = 768
H = D // 2
EPS = 1e-12
_NC, _NS = 2, 16
_NW = _NC * _NS
_CH = 32


def _sc_gather_pack(word_table, idx3):
    nw, n_ch, ch = idx3.shape
    b_total = nw * n_ch * ch
    mesh = plsc.VectorSubcoreMesh(core_axis_name="c", subcore_axis_name="s")

    @functools.partial(
        pl.kernel,
        mesh=mesh,
        out_type=jax.ShapeDtypeStruct((b_total, H), jnp.int32),
        scratch_types=[
            pltpu.VMEM((n_ch, ch), jnp.int32),
            pltpu.VMEM((ch, D), jnp.float32),
            pltpu.VMEM((ch, D), jnp.float32),
            pltpu.VMEM((ch, H), jnp.int32),
            pltpu.VMEM((ch, H), jnp.int32),
            pltpu.SemaphoreType.DMA,
            pltpu.SemaphoreType.DMA,
            pltpu.SemaphoreType.DMA,
            pltpu.SemaphoreType.DMA,
        ],
    )
    def k(table_hbm, idx_hbm, out_hbm, idx_v, gb0, gb1, ob0, ob1,
          gs0, gs1, os0, os1):
        wid = lax.axis_index("s") * _NC + lax.axis_index("c")
        base = wid * (n_ch * ch)
        pltpu.sync_copy(idx_hbm.at[wid], idx_v)
        gbufs = (gb0, gb1)
        obufs = (ob0, ob1)
        gsems = (gs0, gs1)
        osems = (os0, os1)

        pltpu.async_copy(table_hbm.at[idx_v.at[0]], gb0, gs0)
        if n_ch > 1:
            pltpu.async_copy(table_hbm.at[idx_v.at[1]], gb1, gs1)

        def chunk(i, b, c):
            gb, ob = gbufs[b], obufs[b]
            pltpu.make_async_copy(
                table_hbm.at[idx_v.at[c]], gb, gsems[b]).wait()

            @pl.when(i >= 1)
            def _():
                pltpu.make_async_copy(
                    ob, out_hbm.at[pl.ds(base + (c - 2) * ch, ch)],
                    osems[b]).wait()

            def row(r, carry):
                nm = H // 16
                avs = [lax.bitcast_convert_type(gb[r, pl.ds(16 * m, 16)],
                                                jnp.int32) for m in range(nm)]
                bvs = [lax.bitcast_convert_type(gb[r, pl.ds(H + 16 * m, 16)],
                                                jnp.int32) for m in range(nm)]
                for m in range(nm):
                    ob[r, pl.ds(16 * m, 16)] = (
                        lax.shift_right_logical(avs[m], 16)
                        | (bvs[m] & jnp.int32(-65536)))
                return carry

            lax.fori_loop(0, ch, row, 0)

            pltpu.async_copy(ob, out_hbm.at[pl.ds(base + c * ch, ch)],
                             osems[b])

            @pl.when(i < n_ch // 2 - 1)
            def _():
                pltpu.async_copy(table_hbm.at[idx_v.at[c + 2]], gb, gsems[b])

        def body(i, carry):
            chunk(i, 0, 2 * i)
            chunk(i, 1, 2 * i + 1)
            return carry

        lax.fori_loop(0, n_ch // 2, body, 0)

        pltpu.make_async_copy(
            ob0, out_hbm.at[pl.ds(base + (n_ch - 2) * ch, ch)], os0).wait()
        pltpu.make_async_copy(
            ob1, out_hbm.at[pl.ds(base + (n_ch - 1) * ch, ch)], os1).wait()

    return k(word_table, idx3)


def _ln_body(tt_ref, tab_ref, gamma_ref, w_ref, o_ref):
    w = w_ref[...]
    xa = lax.bitcast_convert_type(w << 16, jnp.float32)
    xb = lax.bitcast_convert_type(
        w & jnp.int32(-65536), jnp.float32)
    ttf = tt_ref[0]
    t0a = tab_ref[0, :H][None, :]
    t0b = tab_ref[0, H:][None, :]
    dta = tab_ref[1, :H][None, :] - t0a
    dtb = tab_ref[1, H:][None, :] - t0b
    xa = xa + t0a + ttf * dta
    xb = xb + t0b + ttf * dtb
    mean = (jnp.sum(xa, axis=1, keepdims=True) +
            jnp.sum(xb, axis=1, keepdims=True)) * (1.0 / D)
    msq = (jnp.sum(xa * xa, axis=1, keepdims=True) +
           jnp.sum(xb * xb, axis=1, keepdims=True)) * (1.0 / D)
    rs = lax.rsqrt(msq - mean * mean + EPS)
    bia = mean * rs
    o_ref[:, :H] = (xa * rs - bia) * gamma_ref[0, :H][None, :]
    o_ref[:, H:] = (xb * rs - bia) * gamma_ref[0, H:][None, :]


def _tc_layernorm(packed, token_type_flat, type_table, gamma, tb=2048):
    b_total = packed.shape[0]
    nb = b_total // tb
    tt3 = token_type_flat.reshape(nb, tb, 1).astype(jnp.float32)
    gamma2 = gamma.reshape(1, D)
    return pl.pallas_call(
        _ln_body,
        grid=(nb,),
        in_specs=[
            pl.BlockSpec((1, tb, 1), lambda i: (i, 0, 0)),
            pl.BlockSpec((2, D), lambda i: (0, 0)),
            pl.BlockSpec((1, D), lambda i: (0, 0)),
            pl.BlockSpec((tb, H), lambda i: (i, 0)),
        ],
        out_specs=pl.BlockSpec((tb, D), lambda i: (i, 0)),
        out_shape=jax.ShapeDtypeStruct((b_total, D), jnp.float32),
    )(tt3, type_table, gamma2, packed)


def kernel(input_ids, token_type_ids, word_table, type_table, gamma):
    batch, seq = input_ids.shape
    b_total = batch * seq
    n_ch = b_total // (_NW * _CH)
    idx3 = input_ids.reshape(_NW, n_ch, _CH)
    packed = _sc_gather_pack(word_table, idx3)
    out = _tc_layernorm(packed, token_type_ids.reshape(-1), type_table, gamma)
    return out.reshape(batch, seq, D)
